# baseline (speedup 1.0000x reference)
import functools

import jax
import jax.numpy as jnp
from jax import lax
from jax.experimental import pallas as pl
from jax.experimental.pallas import tpu as pltpu
from jax.experimental.pallas import tpu_sc as plsc

_NC = 2
_NS = 16
_NW = _NC * _NS
_R = 32
_L = 16


def _sc_body(seq_len, x_hbm, pos_hbm, out_hbm, buf, pbuf, sem_in, sem_pos):
    n_rows, H = x_hbm.shape
    rows_per_w = n_rows // _NW
    chunks = rows_per_w // _R
    wid = lax.axis_index("s") * _NC + lax.axis_index("c")
    base = wid * rows_per_w
    w_per_batch = _NW // (n_rows // seq_len)
    seq_base = (wid % w_per_batch) * rows_per_w

    def add_row(r, _):
        for ci in range(H // _L):
            v = pbuf[r, pl.ds(ci * _L, _L)]
            plsc.addupdate(buf.at[r, pl.ds(ci * _L, _L)], v)
        return 0

    def chunk_body(c, _):
        r0 = base + c * _R
        p0 = seq_base + c * _R
        in_cp = pltpu.async_copy(x_hbm.at[pl.ds(r0, _R)], buf, sem_in)
        pos_cp = pltpu.async_copy(pos_hbm.at[pl.ds(p0, _R)], pbuf, sem_pos)
        in_cp.wait()
        pos_cp.wait()
        lax.fori_loop(0, _R, add_row, 0)
        pltpu.sync_copy(buf, out_hbm.at[pl.ds(r0, _R)])
        return 0

    lax.fori_loop(0, chunks, chunk_body, 0)


def kernel(input_tensor, position_embeddings):
    B, S, H = input_tensor.shape
    n_rows = B * S
    x2d = input_tensor.reshape(n_rows, H)

    sc_call = functools.partial(
        pl.kernel,
        out_type=jax.ShapeDtypeStruct((n_rows, H), input_tensor.dtype),
        mesh=plsc.VectorSubcoreMesh(core_axis_name="c", subcore_axis_name="s"),
        scratch_types=[
            pltpu.VMEM((_R, H), input_tensor.dtype),
            pltpu.VMEM((_R, H), input_tensor.dtype),
            pltpu.SemaphoreType.DMA,
            pltpu.SemaphoreType.DMA,
        ],
    )(functools.partial(_sc_body, S))
    out = sc_call(x2d, position_embeddings)
    return out.reshape(B, S, H)


# speedup vs baseline: 2.6812x; 2.6812x over previous
import functools

import jax
import jax.numpy as jnp
from jax import lax
from jax.experimental import pallas as pl
from jax.experimental.pallas import tpu as pltpu
from jax.experimental.pallas import tpu_sc as plsc

_NC = 2
_NS = 16
_NW = _NC * _NS
_R = 8
_L = 16


def _sc_body(seq_len, x_hbm, pos_hbm, out_hbm, buf, pbuf,
             in_sems, pos_sems, out_sems):
    n_rows, H = x_hbm.shape
    B = n_rows // seq_len
    span = seq_len // _NW
    chunks = span // _R
    n_groups = chunks // 2
    wid = lax.axis_index("s") * _NC + lax.axis_index("c")
    seq0 = wid * span

    def in_src(c, b):
        return x_hbm.at[pl.ds(b * seq_len + seq0 + c * _R, _R)]

    def out_dst(c, b):
        return out_hbm.at[pl.ds(b * seq_len + seq0 + c * _R, _R)]

    def pos_src(c):
        return pos_hbm.at[pl.ds(seq0 + c * _R, _R)]

    def fold(s, p):
        def row(r, _):
            for ci in range(H // _L):
                v = pbuf[p, r, pl.ds(ci * _L, _L)]
                plsc.addupdate(buf.at[s, r, pl.ds(ci * _L, _L)], v)
            return 0
        lax.fori_loop(0, _R, row, 0)

    pltpu.async_copy(pos_src(0), pbuf.at[0], pos_sems.at[0])
    pltpu.async_copy(pos_src(1), pbuf.at[1], pos_sems.at[1])
    for j in range(4):
        pltpu.async_copy(in_src(0, j), buf.at[j], in_sems.at[j])

    def group(g, _):
        for j in range(8):
            cj = j // 4
            b = j % 4
            s = j
            c = 2 * g + cj
            if b == 0:
                pltpu.make_async_copy(pos_src(c), pbuf.at[cj],
                                      pos_sems.at[cj]).wait()
            pltpu.make_async_copy(in_src(c, b), buf.at[s],
                                  in_sems.at[s]).wait()
            fold(s, cj)
            pltpu.async_copy(buf.at[s], out_dst(c, b), out_sems.at[s])

            s2 = (j + 4) % 8
            if j < 4:
                @pl.when(g > 0)
                def _():
                    cm = lax.max(c - 1, 0)
                    pltpu.make_async_copy(buf.at[s2], out_dst(cm, b),
                                          out_sems.at[s2]).wait()
                pltpu.async_copy(in_src(c + 1, b), buf.at[s2],
                                 in_sems.at[s2])
            else:
                @pl.when(g < n_groups - 1)
                def _():
                    pltpu.make_async_copy(buf.at[s2], out_dst(c - 1, b),
                                          out_sems.at[s2]).wait()
                    pltpu.async_copy(in_src(c + 1, b), buf.at[s2],
                                     in_sems.at[s2])
            if j == 3 or j == 7:
                @pl.when(g < n_groups - 1)
                def _():
                    pltpu.async_copy(pos_src(c + 2), pbuf.at[cj],
                                     pos_sems.at[cj])
        return 0

    lax.fori_loop(0, n_groups, group, 0)

    for j in range(8):
        c = chunks - 2 + j // 4
        pltpu.make_async_copy(buf.at[j], out_dst(c, j % 4),
                              out_sems.at[j]).wait()


def kernel(input_tensor, position_embeddings):
    B, S, H = input_tensor.shape
    n_rows = B * S
    x2d = input_tensor.reshape(n_rows, H)

    sc_call = functools.partial(
        pl.kernel,
        out_type=jax.ShapeDtypeStruct((n_rows, H), input_tensor.dtype),
        mesh=plsc.VectorSubcoreMesh(core_axis_name="c", subcore_axis_name="s"),
        scratch_types=[
            pltpu.VMEM((8, _R, H), input_tensor.dtype),
            pltpu.VMEM((2, _R, H), input_tensor.dtype),
            pltpu.SemaphoreType.DMA((8,)),
            pltpu.SemaphoreType.DMA((2,)),
            pltpu.SemaphoreType.DMA((8,)),
        ],
    )(functools.partial(_sc_body, S))
    out = sc_call(x2d, position_embeddings)
    return out.reshape(B, S, H)
